# baseline (speedup 1.0000x reference)
import functools

import jax
import jax.numpy as jnp
from jax import lax
from jax.experimental import pallas as pl
from jax.experimental.pallas import tpu as pltpu
from jax.experimental.pallas import tpu_sc as plsc

NUM_VOXEL_X = 128
NUM_VOXEL_Y = 128
NUM_CHANNELS = 80

NC = 2
NS = 16
L = 16
CK = 88


def _sc_pool(ctx3, depth, geom_flat, *, num_cams, num_depth, hw):
    n_slices = num_cams * num_depth
    n_vox = NUM_VOXEL_Y * NUM_VOXEL_X
    CH = NUM_CHANNELS // NC
    spt = n_slices // NS
    rows_per_tile = n_vox // NS
    cps = hw // CK
    ck_offs = [o * L for o in range(CK // L)]
    if CK % L:
        ck_offs.append(CK - L)
    ch_offs = [o * L for o in range(CH // L)]
    if CH % L:
        ch_offs.append(CH - L)

    mesh = plsc.VectorSubcoreMesh(
        core_axis_name="c", subcore_axis_name="s", num_cores=NC, num_subcores=NS
    )

    @functools.partial(
        pl.kernel,
        out_type=jax.ShapeDtypeStruct((n_vox, NUM_CHANNELS), jnp.float32),
        mesh=mesh,
        compiler_params=pltpu.CompilerParams(use_tc_tiling_on_sc=False,
                                             needs_layout_passes=False),
        scratch_types=[
            pltpu.VMEM((CH, hw // 2), jnp.float32),
            pltpu.VMEM((hw, CH), jnp.float32),
            pltpu.VMEM((hw // 2, CH), jnp.float32),
            pltpu.VMEM((hw // 2, CH), jnp.float32),
            pltpu.VMEM((3 * hw,), jnp.int32),
            pltpu.VMEM((3 * hw,), jnp.int32),
            pltpu.VMEM((hw,), jnp.float32),
            pltpu.VMEM((hw,), jnp.float32),
            pltpu.VMEM((cps, CK), jnp.int32),
            pltpu.VMEM((cps, CK), jnp.int32),
            pltpu.VMEM_SHARED((n_vox, CH), jnp.float32),
            pltpu.SemaphoreType.DMA,
            pltpu.SemaphoreType.DMA,
            pltpu.SemaphoreType.DMA,
            pltpu.SemaphoreType.DMA,
        ],
    )
    def pool(ctx_hbm, depth_hbm, geom_hbm, out_hbm,
             stage, table_buf, res_a, res_b, g_a, g_b, d_a, d_b,
             idx_a, idx_b, acc, sem_a, sem_b, lsem_a, lsem_b):
        cid = lax.axis_index("c")
        sid = lax.axis_index("s")

        def zero_row(r, _):
            z = jnp.zeros((L,), jnp.float32)
            for co in ch_offs:
                table_buf[r, pl.ds(co, L)] = z
            return 0

        lax.fori_loop(0, hw, zero_row, 0)
        base_row = sid * rows_per_tile
        pltpu.sync_copy(table_buf, acc.at[pl.ds(base_row, hw), :])
        pltpu.sync_copy(table_buf.at[pl.ds(0, rows_per_tile - hw), :],
                        acc.at[pl.ds(base_row + hw, rows_per_tile - hw), :])
        plsc.subcore_barrier()

        hh = hw // 2
        cph = cps // 2

        lo = sid * spt
        lanes = lax.iota(jnp.int32, L)
        lanes_idx = [jnp.full((L, 1), k, jnp.int32) for k in range(L)]
        gdn = lax.GatherDimensionNumbers(
            offset_dims=(), collapsed_slice_dims=(0,), start_index_map=(0,)
        )

        def bcast_lane(vec, k):
            return lax.gather(vec, lanes_idx[k], dimension_numbers=gdn,
                              slice_sizes=(1,),
                              mode=lax.GatherScatterMode.PROMISE_IN_BOUNDS)

        def load_cam_table(cam):
            for half in range(2):
                pltpu.sync_copy(
                    ctx_hbm.at[cam, pl.ds(cid * CH, CH),
                               pl.ds(half * (hw // 2), hw // 2)],
                    stage,
                )

                def trans_row(r, _):
                    for co in ch_offs:
                        v = plsc.load_gather(
                            stage, [co + lanes, jnp.full((L,), r, jnp.int32)]
                        )
                        table_buf[half * (hw // 2) + r, pl.ds(co, L)] = v
                    return 0

                lax.fori_loop(0, hw // 2, trans_row, 0)

        def mk_idx(g_buf, idx, half):
            def body(r, _):
                for co in ck_offs:
                    p3 = (half * hh + r * CK + co) * 3 + lanes * 3
                    x = plsc.load_gather(g_buf, [p3])
                    y = plsc.load_gather(g_buf, [p3 + 1])
                    x = jnp.minimum(jnp.maximum(x, 0), NUM_VOXEL_X - 1)
                    y = jnp.minimum(jnp.maximum(y, 0), NUM_VOXEL_Y - 1)
                    idx[half * cph + r, pl.ds(co, L)] = y * NUM_VOXEL_X + x
                return 0

            lax.fori_loop(0, cph, body, 0)

        def scale(d_buf, res, half):
            def body(g, _):
                r0 = g * L
                dvec = d_buf[pl.ds(half * hh + r0, L)]
                for k in range(L):
                    dv = bcast_lane(dvec, k)
                    for co in ch_offs:
                        res[r0 + k, pl.ds(co, L)] = (
                            table_buf[half * hh + r0 + k, pl.ds(co, L)] * dv
                        )
                return 0

            lax.fori_loop(0, hh // L, body, 0)

        def fire(res, idx, half, sem):
            for c in range(cph):
                pltpu.async_copy(res.at[pl.ds(c * CK, CK), :],
                                 acc.at[idx.at[half * cph + c]], sem, add=True)

        def drain(res, sem):
            pltpu.make_async_copy(depth_hbm.at[pl.ds(0, hh)],
                                  res, sem).wait()

        def load_slice(s, g_buf, d_buf, lsem):
            pltpu.async_copy(geom_hbm.at[pl.ds(s * 3 * hw, 3 * hw)], g_buf,
                             lsem)
            pltpu.async_copy(depth_hbm.at[pl.ds(s * hw, hw)], d_buf, lsem)

        def load_wait(g_buf, d_buf, lsem):
            pltpu.make_async_copy(geom_hbm.at[pl.ds(0, 3 * hw)], g_buf,
                                  lsem).wait()
            pltpu.make_async_copy(depth_hbm.at[pl.ds(0, hw)], d_buf,
                                  lsem).wait()

        load_slice(lo, g_a, d_a, lsem_a)
        load_slice(lo + 1, g_b, d_b, lsem_b)

        def do_pair(p, prev_cam):
            sa = lo + 2 * p
            cam = sa // num_depth

            @pl.when(cam != prev_cam)
            def _():
                load_cam_table(cam)

            @pl.when(p > 0)
            def _():
                drain(res_a, sem_a)
            load_wait(g_a, d_a, lsem_a)
            mk_idx(g_a, idx_a, 0)
            scale(d_a, res_a, 0)
            fire(res_a, idx_a, 0, sem_a)

            @pl.when(p > 0)
            def _():
                drain(res_b, sem_b)
            mk_idx(g_a, idx_a, 1)
            scale(d_a, res_b, 1)
            fire(res_b, idx_a, 1, sem_b)

            @pl.when(p < spt // 2 - 1)
            def _():
                load_slice(sa + 2, g_a, d_a, lsem_a)

            drain(res_a, sem_a)
            load_wait(g_b, d_b, lsem_b)
            mk_idx(g_b, idx_b, 0)
            scale(d_b, res_a, 0)
            fire(res_a, idx_b, 0, sem_a)

            drain(res_b, sem_b)
            mk_idx(g_b, idx_b, 1)
            scale(d_b, res_b, 1)
            fire(res_b, idx_b, 1, sem_b)

            @pl.when(p < spt // 2 - 1)
            def _():
                load_slice(sa + 3, g_b, d_b, lsem_b)

            return cam

        lax.fori_loop(0, spt // 2, do_pair, jnp.int32(-1))

        drain(res_a, sem_a)
        drain(res_b, sem_b)

        plsc.subcore_barrier()
        pltpu.sync_copy(
            acc.at[pl.ds(base_row, rows_per_tile), :],
            out_hbm.at[pl.ds(base_row, rows_per_tile), pl.ds(cid * CH, CH)],
        )

    return pool(ctx3, depth, geom_flat)


def kernel(geom_xyz, depth_features, context_features,
           batch_size, num_cams, num_depth, num_height, num_width):
    _, cams, _, nh, nw = context_features.shape
    hw = nh * nw
    nd = geom_xyz.shape[0] // (cams * hw)
    ctx3 = context_features.reshape(cams, NUM_CHANNELS, hw).astype(jnp.float32)
    depth = depth_features.astype(jnp.float32)
    geom_flat = geom_xyz.astype(jnp.int32).reshape(-1)

    out = _sc_pool(ctx3, depth, geom_flat, num_cams=cams, num_depth=nd, hw=hw)
    return out.reshape(context_features.shape[0], NUM_VOXEL_Y, NUM_VOXEL_X,
                       NUM_CHANNELS)


# speedup vs baseline: 2.7865x; 2.7865x over previous
import functools

import jax
import jax.numpy as jnp
from jax import lax
from jax.experimental import pallas as pl
from jax.experimental.pallas import tpu as pltpu
from jax.experimental.pallas import tpu_sc as plsc

NUM_VOXEL_X = 128
NUM_VOXEL_Y = 128
NUM_CHANNELS = 80

NC = 2
NS = 16
L = 16
CK = 88


def _sc_pool(table, meta, *, num_cams, num_depth, hw):
    n_slices = num_cams * num_depth
    n_pts = n_slices * hw
    n_vox = NUM_VOXEL_Y * NUM_VOXEL_X
    CH = NUM_CHANNELS // NC
    spt = n_slices // NS
    rows_per_tile = n_vox // NS
    cps = hw // CK
    ck_offs = [o * L for o in range(CK // L)]
    if CK % L:
        ck_offs.append(CK - L)
    ch_offs = [o * L for o in range(CH // L)]
    if CH % L:
        ch_offs.append(CH - L)

    mesh = plsc.VectorSubcoreMesh(
        core_axis_name="c", subcore_axis_name="s", num_cores=NC, num_subcores=NS
    )

    @functools.partial(
        pl.kernel,
        out_type=jax.ShapeDtypeStruct((n_vox, NUM_CHANNELS), jnp.float32),
        mesh=mesh,
        compiler_params=pltpu.CompilerParams(use_tc_tiling_on_sc=False,
                                             needs_layout_passes=False),
        scratch_types=[
            pltpu.VMEM((hw, CH), jnp.float32),
            pltpu.VMEM((hw, CH), jnp.float32),
            pltpu.VMEM((hw, CH), jnp.float32),
            pltpu.VMEM((3, hw), jnp.int32),
            pltpu.VMEM((3, hw), jnp.int32),
            pltpu.VMEM((cps, CK), jnp.int32),
            pltpu.VMEM((cps, CK), jnp.int32),
            pltpu.VMEM_SHARED((n_vox, CH), jnp.float32),
            pltpu.SemaphoreType.DMA,
            pltpu.SemaphoreType.DMA,
            pltpu.SemaphoreType.DMA,
            pltpu.SemaphoreType.DMA,
        ],
    )
    def pool(table_hbm, meta_hbm, out_hbm,
             table_buf, res_a, res_b, pk_a, pk_b, idx_a, idx_b, acc,
             sem_a, sem_b, lsem_a, lsem_b):
        cid = lax.axis_index("c")
        sid = lax.axis_index("s")

        def zero_row(r, _):
            z = jnp.zeros((L,), jnp.float32)
            for co in ch_offs:
                table_buf[r, pl.ds(co, L)] = z
            return 0

        lax.fori_loop(0, hw, zero_row, 0)
        base_row = sid * rows_per_tile
        pltpu.sync_copy(table_buf, acc.at[pl.ds(base_row, hw), :])
        pltpu.sync_copy(table_buf.at[pl.ds(0, rows_per_tile - hw), :],
                        acc.at[pl.ds(base_row + hw, rows_per_tile - hw), :])
        plsc.subcore_barrier()

        lo = sid * spt
        lanes_idx = [jnp.full((L, 1), k, jnp.int32) for k in range(L)]
        gdn = lax.GatherDimensionNumbers(
            offset_dims=(), collapsed_slice_dims=(0,), start_index_map=(0,)
        )

        def bcast_lane(vec, k):
            return lax.gather(vec, lanes_idx[k], dimension_numbers=gdn,
                              slice_sizes=(1,),
                              mode=lax.GatherScatterMode.PROMISE_IN_BOUNDS)

        def mk_idx(pk, idx):
            def body(r, _):
                for co in ck_offs:
                    p = r * CK + co
                    x = pk[1, pl.ds(p, L)]
                    y = pk[2, pl.ds(p, L)]
                    x = jnp.minimum(jnp.maximum(x, 0), NUM_VOXEL_X - 1)
                    y = jnp.minimum(jnp.maximum(y, 0), NUM_VOXEL_Y - 1)
                    idx[r, pl.ds(co, L)] = y * NUM_VOXEL_X + x
                return 0

            lax.fori_loop(0, cps, body, 0)

        def scale(pk, res):
            def body(g, _):
                r0 = g * L
                dvec = plsc.bitcast(pk[0, pl.ds(r0, L)], jnp.float32)
                for k in range(L):
                    dv = bcast_lane(dvec, k)
                    for co in ch_offs:
                        res[r0 + k, pl.ds(co, L)] = (
                            table_buf[r0 + k, pl.ds(co, L)] * dv
                        )
                return 0

            lax.fori_loop(0, hw // L, body, 0)

        def fire(res, idx, sem):
            for c in range(cps):
                pltpu.async_copy(res.at[pl.ds(c * CK, CK), :],
                                 acc.at[idx.at[c]], sem, add=True)

        def drain(res, sem):
            pltpu.make_async_copy(table_hbm.at[cid, pl.ds(0, hw), :],
                                  res, sem).wait()

        def load_slice(s, pk, lsem):
            for part in range(3):
                pltpu.async_copy(
                    meta_hbm.at[pl.ds(part * n_pts + s * hw, hw)],
                    pk.at[part], lsem,
                )

        def load_wait(pk, lsem):
            for part in range(3):
                pltpu.make_async_copy(meta_hbm.at[pl.ds(0, hw)],
                                      pk.at[part], lsem).wait()

        load_slice(lo, pk_a, lsem_a)
        load_slice(lo + 1, pk_b, lsem_b)

        def do_pair(p, prev_cam):
            sa = lo + 2 * p
            cam = sa // num_depth

            @pl.when(cam != prev_cam)
            def _():
                pltpu.sync_copy(
                    table_hbm.at[cid, pl.ds(cam * hw, hw), :], table_buf
                )

            @pl.when(p > 0)
            def _():
                drain(res_a, sem_a)
            load_wait(pk_a, lsem_a)
            mk_idx(pk_a, idx_a)
            scale(pk_a, res_a)
            fire(res_a, idx_a, sem_a)

            @pl.when(p < spt // 2 - 1)
            def _():
                load_slice(sa + 2, pk_a, lsem_a)

            @pl.when(p > 0)
            def _():
                drain(res_b, sem_b)
            load_wait(pk_b, lsem_b)
            mk_idx(pk_b, idx_b)
            scale(pk_b, res_b)
            fire(res_b, idx_b, sem_b)

            @pl.when(p < spt // 2 - 1)
            def _():
                load_slice(sa + 3, pk_b, lsem_b)

            return cam

        lax.fori_loop(0, spt // 2, do_pair, jnp.int32(-1))

        drain(res_a, sem_a)
        drain(res_b, sem_b)

        plsc.subcore_barrier()
        pltpu.sync_copy(
            acc.at[pl.ds(base_row, rows_per_tile), :],
            out_hbm.at[pl.ds(base_row, rows_per_tile), pl.ds(cid * CH, CH)],
        )

    return pool(table, meta)


def kernel(geom_xyz, depth_features, context_features,
           batch_size, num_cams, num_depth, num_height, num_width):
    _, cams, _, nh, nw = context_features.shape
    hw = nh * nw
    nd = geom_xyz.shape[0] // (cams * hw)
    table = jnp.transpose(context_features[0], (0, 2, 3, 1)).reshape(
        cams * hw, NUM_CHANNELS
    ).astype(jnp.float32)
    ch = NUM_CHANNELS // NC
    table2 = jnp.stack([table[:, c * ch:(c + 1) * ch] for c in range(NC)])
    depth_bits = lax.bitcast_convert_type(
        depth_features.astype(jnp.float32), jnp.int32
    )
    geom_i = geom_xyz.astype(jnp.int32)
    meta = jnp.concatenate([depth_bits, geom_i[:, 0], geom_i[:, 1]])

    out = _sc_pool(table2, meta, num_cams=cams, num_depth=nd, hw=hw)
    return out.reshape(context_features.shape[0], NUM_VOXEL_Y, NUM_VOXEL_X,
                       NUM_CHANNELS)
